# Initial kernel scaffold; baseline (speedup 1.0000x reference)
import functools

import jax
import jax.numpy as jnp
from jax import lax
from jax.experimental import pallas as pl
from jax.experimental.pallas import tpu as pltpu
from jax.experimental.pallas import tpu_sc as plsc

NC = 2
NS = 16
LANES = 16
CH = 80


def _spmm_sc(table, cols, vals, rows, n_out_padded):
  E = cols.shape[0]
  D = table.shape[1]
  n_tiles = NC * NS
  assert E % (n_tiles * CH) == 0, (E, n_tiles, CH)
  edges_per_tile = E // n_tiles
  chunks_per_tile = edges_per_tile // CH
  rows_per_tile = n_out_padded // NS
  assert rows_per_tile % CH == 0
  mesh = plsc.VectorSubcoreMesh(core_axis_name="c", subcore_axis_name="s")

  @functools.partial(
      pl.kernel,
      mesh=mesh,
      out_type=jax.ShapeDtypeStruct((NC, n_out_padded, D), jnp.float32),
      scratch_types=[
          pltpu.VMEM((CH,), jnp.int32),
          pltpu.VMEM((CH,), jnp.int32),
          pltpu.VMEM((CH,), jnp.float32),
          pltpu.VMEM((CH, D), jnp.float32),
          pltpu.VMEM_SHARED((n_out_padded, D), jnp.float32),
      ],
  )
  def spmm_kernel(table_h, cols_h, vals_h, rows_h, out_h,
                  cols_v, rows_v, vals_v, gath_v, acc_sh):
    cid = lax.axis_index("c")
    sid = lax.axis_index("s")

    def _zero_row(r, carry):
      for j in range(D // LANES):
        gath_v[r, pl.ds(j * LANES, LANES)] = jnp.zeros((LANES,), jnp.float32)
      return carry
    lax.fori_loop(0, CH, _zero_row, 0)
    for k in range(rows_per_tile // CH):
      pltpu.sync_copy(
          gath_v, acc_sh.at[pl.ds(sid * rows_per_tile + k * CH, CH)])
    plsc.subcore_barrier()

    wid = cid * NS + sid
    tile_base = wid * edges_per_tile

    def _chunk(i, carry):
      base = tile_base + i * CH
      pltpu.sync_copy(cols_h.at[pl.ds(base, CH)], cols_v)
      pltpu.sync_copy(rows_h.at[pl.ds(base, CH)], rows_v)
      pltpu.sync_copy(vals_h.at[pl.ds(base, CH)], vals_v)
      pltpu.sync_copy(table_h.at[cols_v], gath_v)

      def _scale(e, c2):
        vb = plsc.load_gather(vals_v, [jnp.full((LANES,), e, jnp.int32)])
        for j in range(D // LANES):
          sl = pl.ds(j * LANES, LANES)
          gath_v[e, sl] = gath_v[e, sl] * vb
        return c2
      lax.fori_loop(0, CH, _scale, 0)

      pltpu.sync_copy(gath_v, acc_sh.at[rows_v], add=True)
      return carry
    lax.fori_loop(0, chunks_per_tile, _chunk, 0)

    plsc.subcore_barrier()
    pltpu.sync_copy(
        acc_sh.at[pl.ds(sid * rows_per_tile, rows_per_tile)],
        out_h.at[cid, pl.ds(sid * rows_per_tile, rows_per_tile)])

  return spmm_kernel(table, cols, vals, rows)


def _combine2(partials):
  _, S, D = partials.shape
  BR = 512
  assert S % BR == 0

  def body(p_ref, o_ref):
    o_ref[...] = p_ref[0] + p_ref[1]

  return pl.pallas_call(
      body,
      grid=(S // BR,),
      in_specs=[pl.BlockSpec((2, BR, D), lambda i: (0, i, 0))],
      out_specs=pl.BlockSpec((BR, D), lambda i: (i, 0)),
      out_shape=jax.ShapeDtypeStruct((S, D), jnp.float32),
  )(partials)


def _combine_relu_res(partials, x_prev):
  _, S, D = partials.shape
  BR = 512
  assert S % BR == 0

  def body(p_ref, x_ref, o_ref):
    o_ref[...] = jnp.maximum(p_ref[0] + p_ref[1], 0.0) + x_ref[...]

  return pl.pallas_call(
      body,
      grid=(S // BR,),
      in_specs=[
          pl.BlockSpec((2, BR, D), lambda i: (0, i, 0)),
          pl.BlockSpec((BR, D), lambda i: (i, 0)),
      ],
      out_specs=pl.BlockSpec((BR, D), lambda i: (i, 0)),
      out_shape=jax.ShapeDtypeStruct((S, D), jnp.float32),
  )(partials, x_prev)


def _layer_attention_sum(att, xs):
  K, S, D = xs.shape
  BR = 512
  assert S % BR == 0

  def body(att_ref, x_ref, o_ref):
    m = att_ref[0]
    for k in range(1, K):
      m = jnp.maximum(m, att_ref[k])
    es = [jnp.exp(att_ref[k] - m) for k in range(K)]
    denom = sum(es)
    acc = (es[0] / denom) * x_ref[0]
    for k in range(1, K):
      acc = acc + (es[k] / denom) * x_ref[k]
    o_ref[...] = acc

  return pl.pallas_call(
      body,
      grid=(S // BR,),
      in_specs=[
          pl.BlockSpec(memory_space=pltpu.SMEM),
          pl.BlockSpec((K, BR, D), lambda i: (0, i, 0)),
      ],
      out_specs=pl.BlockSpec((BR, D), lambda i: (i, 0)),
      out_shape=jax.ShapeDtypeStruct((S, D), jnp.float32),
  )(att, xs)


@jax.jit
def kernel(pois_embs, tar_rows, tar_cols, tar_vals,
           src_rows, src_cols, src_vals, layer_attention):
  N, D = pois_embs.shape
  L = layer_attention.shape[0] - 1
  H = 1 + int(jnp.zeros(()).shape[0]) if False else 5000
  NP = ((N + 16 * CH - 1) // (16 * CH)) * (16 * CH)
  HP = ((H + 16 * CH - 1) // (16 * CH)) * (16 * CH)

  x = jnp.pad(pois_embs, ((0, NP - N), (0, 0)))
  finals = [x]
  for _ in range(L):
    t_part = _spmm_sc(x, tar_cols, tar_vals, tar_rows, HP)
    t = _combine2(t_part)
    s_part = _spmm_sc(t, src_cols, src_vals, src_rows, NP)
    x = _combine_relu_res(s_part, finals[-1])
    finals.append(x)

  out_p = _layer_attention_sum(layer_attention, jnp.stack(finals))
  return out_p[:N]


# speedup vs baseline: 3.8223x; 3.8223x over previous
import functools

import jax
import jax.numpy as jnp
from jax import lax
from jax.experimental import pallas as pl
from jax.experimental.pallas import tpu as pltpu
from jax.experimental.pallas import tpu_sc as plsc

NC = 2
NS = 16
LANES = 16
CH = 80


def _spmm_sc(table, cols, vals, rows, n_out_padded):
  E = cols.shape[0]
  D = table.shape[1]
  n_tiles = NC * NS
  assert E % (n_tiles * CH) == 0, (E, n_tiles, CH)
  edges_per_tile = E // n_tiles
  chunks_per_tile = edges_per_tile // CH
  rows_per_tile = n_out_padded // NS
  assert rows_per_tile % CH == 0
  mesh = plsc.VectorSubcoreMesh(
      core_axis_name="c", subcore_axis_name="s", num_cores=NC, num_subcores=NS)

  @functools.partial(
      pl.kernel,
      mesh=mesh,
      out_type=jax.ShapeDtypeStruct((NC, n_out_padded, D), jnp.float32),
      scratch_types=[
          pltpu.VMEM((CH,), jnp.int32),
          pltpu.VMEM((CH,), jnp.int32),
          pltpu.VMEM((CH,), jnp.float32),
          pltpu.VMEM((CH, D), jnp.float32),
          pltpu.VMEM_SHARED((n_out_padded, D), jnp.float32),
      ],
  )
  def spmm_kernel(table_h, cols_h, vals_h, rows_h, out_h,
                  cols_v, rows_v, vals_v, gath_v, acc_sh):
    cid = lax.axis_index("c")
    sid = lax.axis_index("s")

    def _zero_row(r, carry):
      for j in range(D // LANES):
        gath_v[r, pl.ds(j * LANES, LANES)] = jnp.zeros((LANES,), jnp.float32)
      return carry
    lax.fori_loop(0, CH, _zero_row, 0)
    for k in range(rows_per_tile // CH):
      pltpu.sync_copy(
          gath_v, acc_sh.at[pl.ds(sid * rows_per_tile + k * CH, CH)])
    plsc.subcore_barrier()

    wid = cid * NS + sid
    tile_base = wid * edges_per_tile

    def _chunk(i, carry):
      base = tile_base + i * CH
      pltpu.sync_copy(cols_h.at[pl.ds(base, CH)], cols_v)
      pltpu.sync_copy(rows_h.at[pl.ds(base, CH)], rows_v)
      pltpu.sync_copy(vals_h.at[pl.ds(base, CH)], vals_v)
      pltpu.sync_copy(table_h.at[cols_v], gath_v)

      def _scale(eb, c2):
        v16 = vals_v[pl.ds(eb * LANES, LANES)]
        for lane in range(LANES):
          e = eb * LANES + lane
          vb = jnp.full((LANES,), v16[lane], jnp.float32)
          for j in range(D // LANES):
            sl = pl.ds(j * LANES, LANES)
            gath_v[e, sl] = gath_v[e, sl] * vb
        return c2
      lax.fori_loop(0, CH // LANES, _scale, 0)

      pltpu.sync_copy(gath_v, acc_sh.at[rows_v], add=True)
      return carry
    lax.fori_loop(0, chunks_per_tile, _chunk, 0)

    plsc.subcore_barrier()
    pltpu.sync_copy(
        acc_sh.at[pl.ds(sid * rows_per_tile, rows_per_tile)],
        out_h.at[cid, pl.ds(sid * rows_per_tile, rows_per_tile)])

  return spmm_kernel(table, cols, vals, rows)


def _combine2(partials):
  _, S, D = partials.shape
  BR = 512
  assert S % BR == 0

  def body(p_ref, o_ref):
    o_ref[...] = p_ref[0] + p_ref[1]

  return pl.pallas_call(
      body,
      grid=(S // BR,),
      in_specs=[pl.BlockSpec((2, BR, D), lambda i: (0, i, 0))],
      out_specs=pl.BlockSpec((BR, D), lambda i: (i, 0)),
      out_shape=jax.ShapeDtypeStruct((S, D), jnp.float32),
  )(partials)


def _combine_relu_res(partials, x_prev):
  _, S, D = partials.shape
  BR = 512
  assert S % BR == 0

  def body(p_ref, x_ref, o_ref):
    o_ref[...] = jnp.maximum(p_ref[0] + p_ref[1], 0.0) + x_ref[...]

  return pl.pallas_call(
      body,
      grid=(S // BR,),
      in_specs=[
          pl.BlockSpec((2, BR, D), lambda i: (0, i, 0)),
          pl.BlockSpec((BR, D), lambda i: (i, 0)),
      ],
      out_specs=pl.BlockSpec((BR, D), lambda i: (i, 0)),
      out_shape=jax.ShapeDtypeStruct((S, D), jnp.float32),
  )(partials, x_prev)


def _layer_attention_sum(att, xs):
  K, S, D = xs.shape
  BR = 512
  assert S % BR == 0

  def body(att_ref, x_ref, o_ref):
    m = att_ref[0]
    for k in range(1, K):
      m = jnp.maximum(m, att_ref[k])
    es = [jnp.exp(att_ref[k] - m) for k in range(K)]
    denom = sum(es)
    acc = (es[0] / denom) * x_ref[0]
    for k in range(1, K):
      acc = acc + (es[k] / denom) * x_ref[k]
    o_ref[...] = acc

  return pl.pallas_call(
      body,
      grid=(S // BR,),
      in_specs=[
          pl.BlockSpec(memory_space=pltpu.SMEM),
          pl.BlockSpec((K, BR, D), lambda i: (0, i, 0)),
      ],
      out_specs=pl.BlockSpec((BR, D), lambda i: (i, 0)),
      out_shape=jax.ShapeDtypeStruct((S, D), jnp.float32),
  )(att, xs)


@jax.jit
def kernel(pois_embs, tar_rows, tar_cols, tar_vals,
           src_rows, src_cols, src_vals, layer_attention):
  N, D = pois_embs.shape
  L = layer_attention.shape[0] - 1
  H = 5000
  NP = ((N + 16 * CH - 1) // (16 * CH)) * (16 * CH)
  HP = ((H + 16 * CH - 1) // (16 * CH)) * (16 * CH)

  x = jnp.pad(pois_embs, ((0, NP - N), (0, 0)))
  finals = [x]
  for _ in range(L):
    t_part = _spmm_sc(x, tar_cols, tar_vals, tar_rows, HP)
    t = _combine2(t_part)
    s_part = _spmm_sc(t, src_cols, src_vals, src_rows, NP)
    x = _combine_relu_res(s_part, finals[-1])
    finals.append(x)

  out_p = _layer_attention_sum(layer_attention, jnp.stack(finals))
  return out_p[:N]
